# baseline (speedup 1.0000x reference)
import jax
import jax.numpy as jnp
from jax.experimental import pallas as pl

_PACK = 4
_ROWS = 6400


def _embed_kernel(x_ref, posb_ref, w_ref, o_ref):
    acc = jax.lax.dot_general(
        x_ref[...], w_ref[...], (((1,), (0,)), ((), ())),
        preferred_element_type=jnp.float32)
    o_ref[...] = acc + posb_ref[...]


def kernel(x, pos_table, W, b):
    B, L, D = x.shape
    Lp = L // _PACK
    Dp = D * _PACK
    rows = B * Lp

    x2 = x.reshape(rows, Dp)
    posb = jnp.tile(pos_table.reshape(Lp, Dp) + jnp.tile(b, _PACK)[None, :],
                    (_ROWS // Lp, 1))

    wd = jnp.zeros((Dp, Dp), dtype=W.dtype)
    for i in range(_PACK):
        wd = wd.at[i * D:(i + 1) * D, i * D:(i + 1) * D].set(W)

    out = pl.pallas_call(
        _embed_kernel,
        grid=(rows // _ROWS,),
        in_specs=[
            pl.BlockSpec((_ROWS, Dp), lambda i: (i, 0)),
            pl.BlockSpec((_ROWS, Dp), lambda i: (0, 0)),
            pl.BlockSpec((Dp, Dp), lambda i: (0, 0)),
        ],
        out_specs=pl.BlockSpec((_ROWS, Dp), lambda i: (i, 0)),
        out_shape=jax.ShapeDtypeStruct((rows, Dp), x.dtype),
    )(x2, posb, wd)
    return out.reshape(B, L, D)


# speedup vs baseline: 1.0816x; 1.0816x over previous
import jax
import jax.numpy as jnp
from jax.experimental import pallas as pl

_BB = 64


def _embed_kernel(x_ref, posb_ref, w_ref, o_ref):
    acc = jax.lax.dot_general(
        x_ref[...], w_ref[...], (((2,), (0,)), ((), ())),
        preferred_element_type=jnp.float32)
    o_ref[...] = acc + posb_ref[...][None, :, :]


def kernel(x, pos_table, W, b):
    B, L, D = x.shape
    posb = pos_table + b[None, :]

    return pl.pallas_call(
        _embed_kernel,
        grid=(B // _BB,),
        in_specs=[
            pl.BlockSpec((_BB, L, D), lambda i: (i, 0, 0)),
            pl.BlockSpec((L, D), lambda i: (0, 0)),
            pl.BlockSpec((D, D), lambda i: (0, 0)),
        ],
        out_specs=pl.BlockSpec((_BB, L, D), lambda i: (i, 0, 0)),
        out_shape=jax.ShapeDtypeStruct((B, L, D), x.dtype),
    )(x, posb, W)
